# baseline (speedup 1.0000x reference)
import functools

import jax
import jax.numpy as jnp
from jax import lax
from jax.experimental import pallas as pl
from jax.experimental.pallas import tpu as pltpu
from jax.experimental.pallas import tpu_sc as plsc

_NEGATIVE_RATIO = 3.0
_EPS = 1e-06
_NBINS = 22528
_NSUB = 32


def _f16_bits(v):
    u = lax.bitcast_convert_type(v, jnp.int32)
    add = 0xFFF + (lax.shift_right_logical(u, 13) & 1)
    normal = lax.shift_right_logical(u + add, 13) - (112 << 10)
    scaled = v * 16777216.0
    fl = scaled.astype(jnp.int32)
    frac = scaled - fl.astype(jnp.float32)
    round_up = (frac > 0.5) | ((frac == 0.5) & ((fl & 1) == 1))
    sub = fl + round_up.astype(jnp.int32)
    return jnp.where(v < 6.103515625e-05, sub, normal)


def _stage1_body(p_ref, g_ref, m_ref, packed_ref, pos_ref, neg_ref, psum_ref):
    step = pl.program_id(0)
    p = p_ref[...]
    g = g_ref[...]
    m = m_ref[...]
    n = p.shape[0]
    big_g = jnp.sum(g, axis=0, keepdims=True)
    sel = jnp.where(g > 0.5, p, 1.0 - p)
    loss = -jnp.maximum(jnp.log(sel), -100.0)
    v = loss * m
    wneg = float(n) - big_g
    packed_ref[...] = _f16_bits(v) | lax.shift_left(
        wneg.astype(jnp.int32), 16)

    @pl.when(step == 0)
    def _():
        pos_ref[0, 0] = 0.0
        neg_ref[0, 0] = 0.0
        psum_ref[0, 0] = 0.0

    pos_ref[0, 0] += jnp.sum(m * big_g)
    neg_ref[0, 0] += jnp.sum(m * wneg)
    psum_ref[0, 0] += jnp.sum(v * big_g)


def _stage1(p3, g3, mask, h0, hh, interpret=False):
    n, h, w = p3.shape
    rows = 64
    grid = hh // rows
    off = h0 // rows
    bspec = pl.BlockSpec((n, rows, w), lambda i: (0, i + off, 0))
    ospec = pl.BlockSpec((n, rows, w), lambda i: (0, i, 0))
    sspec = pl.BlockSpec(memory_space=pltpu.SMEM, block_shape=(1, 1),
                         index_map=lambda i: (0, 0))
    return pl.pallas_call(
        _stage1_body,
        grid=(grid,),
        in_specs=[bspec, bspec, bspec],
        out_specs=[ospec, sspec, sspec, sspec],
        out_shape=[
            jax.ShapeDtypeStruct((n, hh, w), jnp.int32),
            jax.ShapeDtypeStruct((1, 1), jnp.float32),
            jax.ShapeDtypeStruct((1, 1), jnp.float32),
            jax.ShapeDtypeStruct((1, 1), jnp.float32),
        ],
        interpret=interpret,
    )(p3, g3, mask)


def _sc_hist(packed3):
    n, h, w = packed3.shape
    rows_per_sub = (n * h) // _NSUB
    chunk = rows_per_sub * w
    hrows = _NBINS // 128
    subs_per_n = h // rows_per_sub

    mesh = plsc.VectorSubcoreMesh(core_axis_name="c", subcore_axis_name="s")

    @functools.partial(
        pl.kernel,
        mesh=mesh,
        out_type=jax.ShapeDtypeStruct((_NSUB, hrows, 128), jnp.float32),
        scratch_types=[
            pltpu.VMEM((rows_per_sub, w), jnp.int32),
            pltpu.VMEM((hrows, 128), jnp.float32),
            pltpu.SemaphoreType.DMA,
        ],
        compiler_params=pltpu.CompilerParams(needs_layout_passes=False),
    )
    def hist_kernel(packed_hbm, out_hbm, pk_v, hist_v, sem):
        wid = lax.axis_index("c") * 16 + lax.axis_index("s")
        j = wid // subs_per_n
        r0 = (wid % subs_per_n) * rows_per_sub
        copy_in = pltpu.async_copy(
            packed_hbm.at[j, pl.ds(r0, rows_per_sub)], pk_v, sem)

        zeros = jnp.zeros((16,), jnp.float32)

        @plsc.parallel_loop(0, _NBINS, step=16, unroll=8)
        def zero_body(i):
            hist_v[i // 128, pl.ds(lax.rem(i, 128), 16)] = zeros

        copy_in.wait()

        @plsc.parallel_loop(0, chunk, step=16, unroll=16)
        def scat_body(i):
            pk = pk_v[i // w, pl.ds(lax.rem(i, w), 16)]
            bins = pk & 0xFFFF
            wgt = lax.shift_right_logical(pk, 16).astype(jnp.float32)
            live = (bins != 0) & (wgt > 0.0)
            plsc.addupdate_scatter(
                hist_v,
                [lax.shift_right_logical(bins, 7), bins & 127],
                wgt, mask=live)

        pltpu.sync_copy(hist_v, out_hbm.at[wid])

    return hist_kernel(packed3)


def _fin_body(ha_ref, hb_ref, pos_a_ref, neg_a_ref, psum_a_ref,
              pos_b_ref, neg_b_ref, psum_b_ref, out_ref):
    cnt = jnp.sum(ha_ref[...], axis=0) + jnp.sum(hb_ref[...], axis=0)
    r128 = cnt.shape[0]
    iu0 = lax.broadcasted_iota(jnp.int32, (128, 128), 0)
    iu1 = lax.broadcasted_iota(jnp.int32, (128, 128), 1)
    upper = (iu0 <= iu1).astype(jnp.float32)
    row_cum = jnp.dot(cnt, upper, preferred_element_type=jnp.float32)
    rtot = jnp.sum(cnt, axis=1, keepdims=True)
    il0 = lax.broadcasted_iota(jnp.int32, (r128, r128), 0)
    il1 = lax.broadcasted_iota(jnp.int32, (r128, r128), 1)
    lower_strict = (il0 > il1).astype(jnp.float32)
    row_off = jnp.dot(lower_strict, jnp.broadcast_to(rtot, cnt.shape),
                      preferred_element_type=jnp.float32)
    prefix = row_cum + row_off
    total = jnp.sum(cnt)
    above = total - prefix
    incl = above + cnt

    pidx = (lax.broadcasted_iota(jnp.int32, cnt.shape, 0) * 128
            + lax.broadcasted_iota(jnp.int32, cnt.shape, 1))
    exp_mant = (lax.shift_left(lax.shift_right_logical(pidx, 10) + 112, 23)
                | lax.shift_left(pidx & 1023, 13))
    val_norm = lax.bitcast_convert_type(exp_mant, jnp.float32)
    val = jnp.where(pidx < 1024, pidx.astype(jnp.float32) * (2.0 ** -24),
                    val_norm)

    pos = pos_a_ref[0, 0] + pos_b_ref[0, 0]
    neg = neg_a_ref[0, 0] + neg_b_ref[0, 0]
    psum = psum_a_ref[0, 0] + psum_b_ref[0, 0]
    c = jnp.minimum(neg, jnp.floor(pos * _NEGATIVE_RATIO))
    full = incl <= c
    part = (above < c) & (incl > c)
    neg_sum = (jnp.sum(jnp.where(full, cnt * val, 0.0))
               + jnp.sum(jnp.where(part, (c - above) * val, 0.0)))
    out_ref[0, 0] = (psum + neg_sum) / (pos + c + _EPS)


def _finalize(ha, hb, scalars, interpret=False):
    sspec = pl.BlockSpec(memory_space=pltpu.SMEM)
    return pl.pallas_call(
        _fin_body,
        in_specs=[pl.BlockSpec(ha.shape, lambda: (0, 0, 0)),
                  pl.BlockSpec(hb.shape, lambda: (0, 0, 0)),
                  sspec, sspec, sspec, sspec, sspec, sspec],
        out_specs=pl.BlockSpec(memory_space=pltpu.SMEM),
        out_shape=jax.ShapeDtypeStruct((1, 1), jnp.float32),
        interpret=interpret,
    )(ha, hb, *scalars)


def kernel(pred, gt, mask):
    n, _, h, w = pred.shape
    p3 = pred.reshape(n, h, w)
    g3 = gt.reshape(n, h, w)
    half = h // 2
    pk_a, pos_a, neg_a, ps_a = _stage1(p3, g3, mask, 0, half)
    hist_a = _sc_hist(pk_a)
    pk_b, pos_b, neg_b, ps_b = _stage1(p3, g3, mask, half, half)
    hist_b = _sc_hist(pk_b)
    out = _finalize(hist_a, hist_b,
                    (pos_a, neg_a, ps_a, pos_b, neg_b, ps_b))
    return out.reshape(())


# speedup vs baseline: 1.0761x; 1.0761x over previous
import functools

import jax
import jax.numpy as jnp
from jax import lax
from jax.experimental import pallas as pl
from jax.experimental.pallas import tpu as pltpu
from jax.experimental.pallas import tpu_sc as plsc

_NEGATIVE_RATIO = 3.0
_EPS = 1e-06
_NBINS = 22528
_NSUB = 32


def _f16_bits(v):
    u = lax.bitcast_convert_type(v, jnp.int32)
    add = 0xFFF + (lax.shift_right_logical(u, 13) & 1)
    normal = lax.shift_right_logical(u + add, 13) - (112 << 10)
    scaled = v * 16777216.0
    fl = scaled.astype(jnp.int32)
    frac = scaled - fl.astype(jnp.float32)
    round_up = (frac > 0.5) | ((frac == 0.5) & ((fl & 1) == 1))
    sub = fl + round_up.astype(jnp.int32)
    return jnp.where(v < 6.103515625e-05, sub, normal)


def _stage1_body(p_ref, g_ref, m_ref, packed_ref, pos_ref, neg_ref, psum_ref):
    step = pl.program_id(0)
    p = p_ref[...]
    g = g_ref[...]
    m = m_ref[...]
    n = p.shape[0]
    big_g = jnp.sum(g, axis=0, keepdims=True)
    sel = jnp.where(g > 0.5, p, 1.0 - p)
    loss = -jnp.maximum(jnp.log(sel), -100.0)
    v = loss * m
    wneg = float(n) - big_g
    packed_ref[...] = _f16_bits(v) | lax.shift_left(
        wneg.astype(jnp.int32), 16)

    @pl.when(step == 0)
    def _():
        pos_ref[0, 0] = 0.0
        neg_ref[0, 0] = 0.0
        psum_ref[0, 0] = 0.0

    pos_ref[0, 0] += jnp.sum(m * big_g)
    neg_ref[0, 0] += jnp.sum(m * wneg)
    psum_ref[0, 0] += jnp.sum(v * big_g)


def _stage1(p3, g3, mask, interpret=False):
    n, h, w = p3.shape
    rows = 64
    grid = h // rows
    bspec = pl.BlockSpec((n, rows, w), lambda i: (0, i, 0))
    sspec = pl.BlockSpec(memory_space=pltpu.SMEM, block_shape=(1, 1),
                         index_map=lambda i: (0, 0))
    return pl.pallas_call(
        _stage1_body,
        grid=(grid,),
        in_specs=[bspec, bspec, bspec],
        out_specs=[bspec, sspec, sspec, sspec],
        out_shape=[
            jax.ShapeDtypeStruct((n, h, w), jnp.int32),
            jax.ShapeDtypeStruct((1, 1), jnp.float32),
            jax.ShapeDtypeStruct((1, 1), jnp.float32),
            jax.ShapeDtypeStruct((1, 1), jnp.float32),
        ],
        interpret=interpret,
    )(p3, g3, mask)


def _sc_hist(packed3):
    n, h, w = packed3.shape
    rows_per_sub = (n * h) // _NSUB
    chunk = rows_per_sub * w
    hrows = _NBINS // 128
    subs_per_n = h // rows_per_sub

    mesh = plsc.VectorSubcoreMesh(core_axis_name="c", subcore_axis_name="s")

    @functools.partial(
        pl.kernel,
        mesh=mesh,
        out_type=jax.ShapeDtypeStruct((_NSUB, hrows, 128), jnp.float32),
        scratch_types=[
            pltpu.VMEM((rows_per_sub, w), jnp.int32),
            pltpu.VMEM((hrows, 128), jnp.float32),
            pltpu.SemaphoreType.DMA,
        ],
        compiler_params=pltpu.CompilerParams(needs_layout_passes=False),
    )
    def hist_kernel(packed_hbm, out_hbm, pk_v, hist_v, sem):
        wid = lax.axis_index("c") * 16 + lax.axis_index("s")
        j = wid // subs_per_n
        r0 = (wid % subs_per_n) * rows_per_sub
        copy_in = pltpu.async_copy(
            packed_hbm.at[j, pl.ds(r0, rows_per_sub)], pk_v, sem)

        zeros = jnp.zeros((16,), jnp.float32)

        @plsc.parallel_loop(0, _NBINS, step=16, unroll=8)
        def zero_body(i):
            hist_v[i // 128, pl.ds(lax.rem(i, 128), 16)] = zeros

        copy_in.wait()

        @plsc.parallel_loop(0, chunk, step=16, unroll=16)
        def scat_body(i):
            pk = pk_v[i // w, pl.ds(lax.rem(i, w), 16)]
            bins = pk & 0xFFFF
            wgt = lax.shift_right_logical(pk, 16).astype(jnp.float32)
            live = (bins != 0) & (wgt > 0.0)
            plsc.addupdate_scatter(
                hist_v,
                [lax.shift_right_logical(bins, 7), bins & 127],
                wgt, mask=live)

        pltpu.sync_copy(hist_v, out_hbm.at[wid])

    return hist_kernel(packed3)


def _fin_body(h_ref, pos_ref, neg_ref, psum_ref, out_ref):
    h3 = h_ref[...]
    cnt = jnp.sum(h3, axis=0)
    r128 = cnt.shape[0]
    iu0 = lax.broadcasted_iota(jnp.int32, (128, 128), 0)
    iu1 = lax.broadcasted_iota(jnp.int32, (128, 128), 1)
    upper = (iu0 <= iu1).astype(jnp.float32)
    row_cum = jnp.dot(cnt, upper, preferred_element_type=jnp.float32)
    rtot = jnp.sum(cnt, axis=1, keepdims=True)
    il0 = lax.broadcasted_iota(jnp.int32, (r128, r128), 0)
    il1 = lax.broadcasted_iota(jnp.int32, (r128, r128), 1)
    lower_strict = (il0 > il1).astype(jnp.float32)
    row_off = jnp.dot(lower_strict, jnp.broadcast_to(rtot, cnt.shape),
                      preferred_element_type=jnp.float32)
    prefix = row_cum + row_off
    total = jnp.sum(cnt)
    above = total - prefix
    incl = above + cnt

    pidx = (lax.broadcasted_iota(jnp.int32, cnt.shape, 0) * 128
            + lax.broadcasted_iota(jnp.int32, cnt.shape, 1))
    exp_mant = (lax.shift_left(lax.shift_right_logical(pidx, 10) + 112, 23)
                | lax.shift_left(pidx & 1023, 13))
    val_norm = lax.bitcast_convert_type(exp_mant, jnp.float32)
    val = jnp.where(pidx < 1024, pidx.astype(jnp.float32) * (2.0 ** -24),
                    val_norm)

    pos = pos_ref[0, 0]
    neg = neg_ref[0, 0]
    psum = psum_ref[0, 0]
    c = jnp.minimum(neg, jnp.floor(pos * _NEGATIVE_RATIO))
    full = incl <= c
    part = (above < c) & (incl > c)
    neg_sum = (jnp.sum(jnp.where(full, cnt * val, 0.0))
               + jnp.sum(jnp.where(part, (c - above) * val, 0.0)))
    out_ref[0, 0] = (psum + neg_sum) / (pos + c + _EPS)


def _finalize(hists3, pos, neg, psum, interpret=False):
    sspec = pl.BlockSpec(memory_space=pltpu.SMEM)
    return pl.pallas_call(
        _fin_body,
        in_specs=[pl.BlockSpec(hists3.shape, lambda: (0, 0, 0)),
                  sspec, sspec, sspec],
        out_specs=pl.BlockSpec(memory_space=pltpu.SMEM),
        out_shape=jax.ShapeDtypeStruct((1, 1), jnp.float32),
        interpret=interpret,
    )(hists3, pos, neg, psum)


def kernel(pred, gt, mask):
    n, _, h, w = pred.shape
    p3 = pred.reshape(n, h, w)
    g3 = gt.reshape(n, h, w)
    packed, pos, neg, psum = _stage1(p3, g3, mask)
    hists = _sc_hist(packed)
    out = _finalize(hists, pos, neg, psum)
    return out.reshape(())
